# speedup vs baseline: 2.0195x; 2.0195x over previous
import functools

import jax
import jax.numpy as jnp
from jax import lax
from jax.experimental import pallas as pl
from jax.experimental.pallas import tpu as pltpu
from jax.experimental.pallas import tpu_sc as plsc

H = 128
NC = 2
NS = 16
NW = NC * NS
CK = 128
BIGIDX = 1 << 30


def _mesh():
  return plsc.VectorSubcoreMesh(
      core_axis_name="c", subcore_axis_name="s", num_cores=NC,
      num_subcores=NS)


@functools.partial(jax.jit, static_argnums=(4,))
def _sc_gather(xs, xr, sidx, ridx, ep):
  nch = ep // NW // CK

  @functools.partial(
      pl.kernel,
      mesh=_mesh(),
      out_type=(
          jax.ShapeDtypeStruct((ep, H), jnp.float32),
          jax.ShapeDtypeStruct((ep, H), jnp.float32),
      ),
      scratch_types=[
          pltpu.VMEM((CK,), jnp.int32),
          pltpu.VMEM((CK,), jnp.int32),
          pltpu.VMEM((CK, H), jnp.float32),
          pltpu.VMEM((CK, H), jnp.float32),
          pltpu.SemaphoreType.DMA,
          pltpu.SemaphoreType.DMA,
      ],
  )
  def k(xs_hbm, xr_hbm, sidx_hbm, ridx_hbm, outs_hbm, outr_hbm,
        si_v, ri_v, srow_v, rrow_v, sem0, sem1):
    wid = lax.axis_index("s") * NC + lax.axis_index("c")
    base = wid * (nch * CK)

    def body(ci, _):
      off = base + ci * CK
      pltpu.sync_copy(sidx_hbm.at[pl.ds(off, CK)], si_v)
      pltpu.sync_copy(ridx_hbm.at[pl.ds(off, CK)], ri_v)
      g0 = pltpu.async_copy(xs_hbm.at[si_v], srow_v, sem0)
      g1 = pltpu.async_copy(xr_hbm.at[ri_v], rrow_v, sem1)
      g0.wait()
      g1.wait()
      pltpu.sync_copy(srow_v, outs_hbm.at[pl.ds(off, CK)])
      pltpu.sync_copy(rrow_v, outr_hbm.at[pl.ds(off, CK)])
      return ()

    lax.fori_loop(0, nch, body, ())

  return k(xs, xr, sidx, ridx)


@functools.partial(jax.jit, static_argnums=(2, 3))
def _sc_scatter(msg, ridx, ep, nrec):
  nh = nrec // NC
  accr = ((nh + 64) + 1023) // 1024 * 1024
  rpt = accr // NS
  nzc = rpt // 64
  nch = ep // NS // CK
  ock = 200
  noc = nh // ock
  nopt = (noc + NS - 1) // NS

  @functools.partial(
      pl.kernel,
      mesh=_mesh(),
      out_type=jax.ShapeDtypeStruct((nrec, H), jnp.float32),
      scratch_types=[
          pltpu.VMEM((64, H), jnp.float32),
          pltpu.VMEM((CK,), jnp.int32),
          pltpu.VMEM((CK,), jnp.int32),
          pltpu.VMEM((CK, H), jnp.float32),
          pltpu.VMEM_SHARED((accr, H), jnp.float32),
      ],
  )
  def k(msg_hbm, ridx_hbm, out_hbm, zb_v, ri_v, li_v, m_v, acc):
    c = lax.axis_index("c")
    t = lax.axis_index("s")

    def zb(i, _):
      zb_v[i // 8, pl.ds((i % 8) * 16, 16)] = jnp.zeros((16,), jnp.float32)
      return ()
    lax.fori_loop(0, 64 * 8, zb, ())

    zbase = t * rpt
    def zc(i, _):
      pltpu.sync_copy(zb_v, acc.at[pl.ds(zbase + i * 64, 64)])
      return ()
    lax.fori_loop(0, nzc, zc, ())
    plsc.subcore_barrier()

    lo = c * nh
    base = t * (nch * CK)

    def body(ci, _):
      off = base + ci * CK
      pltpu.sync_copy(ridx_hbm.at[pl.ds(off, CK)], ri_v)
      def fix(j, _):
        v = ri_v[pl.ds(j * 16, 16)]
        lv = v - lo
        ok = (lv >= 0) & (lv < nh)
        li_v[pl.ds(j * 16, 16)] = jnp.where(ok, lv, nh)
        return ()
      lax.fori_loop(0, CK // 16, fix, ())
      pltpu.sync_copy(msg_hbm.at[pl.ds(off, CK)], m_v)
      pltpu.sync_copy(m_v, acc.at[li_v], add=True)
      return ()

    lax.fori_loop(0, nch, body, ())
    plsc.subcore_barrier()

    def oc(i, _):
      ch = t + i * NS
      @pl.when(ch < noc)
      def _():
        pltpu.sync_copy(acc.at[pl.ds(ch * ock, ock)],
                        out_hbm.at[pl.ds(lo + ch * ock, ock)])
      return ()
    lax.fori_loop(0, nopt, oc, ())

  return k(msg, ridx)


def _edge_body(send_b, rec_b, invt_b, w1s_b, w1r_b, w1i_b, b1_b, w2_b, b2_b,
               ewt_b, eb_b, out_b):
  f32 = jnp.float32
  h = (jnp.dot(send_b[...], w1s_b[...], preferred_element_type=f32)
       + jnp.dot(rec_b[...], w1r_b[...], preferred_element_type=f32)
       + lax.dot_general(invt_b[...], w1i_b[...],
                         (((0,), (0,)), ((), ())), preferred_element_type=f32)
       + b1_b[...])
  h = h * jax.nn.sigmoid(h)
  m = jnp.dot(h, w2_b[...], preferred_element_type=f32) + b2_b[...]
  m = m * jax.nn.sigmoid(m)
  wg = jax.nn.sigmoid(jnp.sum(m * ewt_b[...], axis=1, keepdims=True)
                      + eb_b[0])
  out_b[...] = m * wg


@functools.partial(jax.jit, static_argnums=(11,))
def _edge_mlp(send, rec, invt, w1s, w1r, w1i, b1, w2, b2, ewt, eb, ep):
  be = 1024
  grid = (ep // be,)
  wspec = pl.BlockSpec((H, H), lambda i: (0, 0))
  bspec = pl.BlockSpec((1, H), lambda i: (0, 0))
  return pl.pallas_call(
      _edge_body,
      grid=grid,
      in_specs=[
          pl.BlockSpec((be, H), lambda i: (i, 0)),
          pl.BlockSpec((be, H), lambda i: (i, 0)),
          pl.BlockSpec((8, be), lambda i: (0, i)),
          wspec, wspec,
          pl.BlockSpec((8, H), lambda i: (0, 0)),
          bspec, wspec, bspec, bspec,
          pl.BlockSpec(memory_space=pltpu.SMEM),
      ],
      out_specs=pl.BlockSpec((be, H), lambda i: (i, 0)),
      out_shape=jax.ShapeDtypeStruct((ep, H), jnp.float32),
  )(send, rec, invt, w1s, w1r, w1i, b1, w2, b2, ewt, eb)


def _node0_body(x_b, m_b, a_b, bq_b, b1_b, w2_b, b2_b, sw_b, sb_b, out_b):
  f32 = jnp.float32
  h = (jnp.dot(x_b[...], a_b[...], preferred_element_type=f32)
       + jnp.dot(m_b[...], bq_b[...], preferred_element_type=f32)
       + b1_b[...])
  h = h * jax.nn.sigmoid(h)
  out_b[...] = (jnp.dot(h, w2_b[...], preferred_element_type=f32) + b2_b[...]
                + jnp.dot(x_b[...], sw_b[...], preferred_element_type=f32)
                + sb_b[...])


def _node1_body(x_b, m1_b, m2_b, a_b, bq_b, c_b, b1_b, w2_b, b2_b, sw_b,
                sb_b, out_b):
  f32 = jnp.float32
  h = (jnp.dot(x_b[...], a_b[...], preferred_element_type=f32)
       + jnp.dot(m1_b[...], bq_b[...], preferred_element_type=f32)
       + jnp.dot(m2_b[...], c_b[...], preferred_element_type=f32)
       + b1_b[...])
  h = h * jax.nn.sigmoid(h)
  out_b[...] = (jnp.dot(h, w2_b[...], preferred_element_type=f32) + b2_b[...]
                + jnp.dot(x_b[...], sw_b[...], preferred_element_type=f32)
                + sb_b[...])


@jax.jit
def _node0(x, m, a, bq, b1, w2, b2, sw, sb):
  n = x.shape[0]
  bn = 1000
  wspec = pl.BlockSpec((H, H), lambda i: (0, 0))
  bspec = pl.BlockSpec((1, H), lambda i: (0, 0))
  nspec = pl.BlockSpec((bn, H), lambda i: (i, 0))
  return pl.pallas_call(
      _node0_body,
      grid=(n // bn,),
      in_specs=[nspec, nspec, wspec, wspec, bspec, wspec, bspec, wspec,
                bspec],
      out_specs=nspec,
      out_shape=jax.ShapeDtypeStruct((n, H), jnp.float32),
  )(x, m, a, bq, b1, w2, b2, sw, sb)


@jax.jit
def _node1(x, m1, m2, a, bq, c, b1, w2, b2, sw, sb):
  n = x.shape[0]
  bn = 1000
  wspec = pl.BlockSpec((H, H), lambda i: (0, 0))
  bspec = pl.BlockSpec((1, H), lambda i: (0, 0))
  nspec = pl.BlockSpec((bn, H), lambda i: (i, 0))
  return pl.pallas_call(
      _node1_body,
      grid=(n // bn,),
      in_specs=[nspec, nspec, nspec, wspec, wspec, wspec, bspec, wspec,
                bspec, wspec, bspec],
      out_specs=nspec,
      out_shape=jax.ShapeDtypeStruct((n, H), jnp.float32),
  )(x, m1, m2, a, bq, c, b1, w2, b2, sw, sb)


def _pad_to(x, n, val):
  e = x.shape[0]
  if e == n:
    return x
  return jnp.concatenate(
      [x, jnp.full((n - e,) + x.shape[1:], val, x.dtype)], axis=0)


def _message(xs, xr, adj, inv, w1, b1, w2, b2, ew, eb, nrec):
  e = adj.shape[1]
  ep = -(-e // (NW * CK)) * (NW * CK)
  ninv = inv.shape[1]
  sidx = _pad_to(adj[0].astype(jnp.int32), ep, 0)
  gidx = _pad_to(adj[1].astype(jnp.int32), ep, 0)
  ridx = _pad_to(adj[1].astype(jnp.int32), ep, BIGIDX)
  invt = jnp.zeros((8, ep), jnp.float32).at[:ninv, :e].set(inv.T)
  w1s = w1[:H]
  w1r = w1[H:2 * H]
  w1i = jnp.zeros((8, H), jnp.float32).at[:ninv].set(w1[2 * H:])
  send, rec = _sc_gather(xs, xr, sidx, gidx, ep)
  msg = _edge_mlp(send, rec, invt, w1s, w1r, w1i, b1.reshape(1, H), w2,
                  b2.reshape(1, H), ew.reshape(1, H), eb, ep)
  return _sc_scatter(msg, ridx, ep, nrec)


def kernel(x0, x1, adj_0_0, adj_0_1, adj_1_1, inv_0_0, inv_0_1, inv_1_1,
           mw1_0_0, mb1_0_0, mw2_0_0, mb2_0_0, ew_0_0, eb_0_0,
           mw1_0_1, mb1_0_1, mw2_0_1, mb2_0_1, ew_0_1, eb_0_1,
           mw1_1_1, mb1_1_1, mw2_1_1, mb2_1_1, ew_1_1, eb_1_1,
           u0w1, u0b1, u0w2, u0b2, u1w1, u1b1, u1w2, u1b2, sw, sb):
  m00 = _message(x0, x0, adj_0_0, inv_0_0, mw1_0_0, mb1_0_0, mw2_0_0,
                 mb2_0_0, ew_0_0, eb_0_0, x0.shape[0])
  m01 = _message(x0, x1, adj_0_1, inv_0_1, mw1_0_1, mb1_0_1, mw2_0_1,
                 mb2_0_1, ew_0_1, eb_0_1, x1.shape[0])
  m11 = _message(x1, x1, adj_1_1, inv_1_1, mw1_1_1, mb1_1_1, mw2_1_1,
                 mb2_1_1, ew_1_1, eb_1_1, x1.shape[0])
  out0 = _node0(x0, m00, u0w1[:H], u0w1[H:], u0b1.reshape(1, H), u0w2,
                u0b2.reshape(1, H), sw, sb.reshape(1, H))
  out1 = _node1(x1, m01, m11, u1w1[:H], u1w1[H:2 * H], u1w1[2 * H:],
                u1b1.reshape(1, H), u1w2, u1b2.reshape(1, H), sw,
                sb.reshape(1, H))
  return (out0, out1)


# speedup vs baseline: 2.0324x; 1.0064x over previous
import functools

import jax
import jax.numpy as jnp
from jax import lax
from jax.experimental import pallas as pl
from jax.experimental.pallas import tpu as pltpu
from jax.experimental.pallas import tpu_sc as plsc

H = 128
NC = 2
NS = 16
NW = NC * NS
CK = 128
BIGIDX = 1 << 30


def _mesh():
  return plsc.VectorSubcoreMesh(
      core_axis_name="c", subcore_axis_name="s", num_cores=NC,
      num_subcores=NS)


@functools.partial(jax.jit, static_argnums=(4,))
def _sc_gather(xs, xr, sidx, ridx, ep):
  nch = ep // NW // CK
  assert nch % 2 == 0

  @functools.partial(
      pl.kernel,
      mesh=_mesh(),
      out_type=(
          jax.ShapeDtypeStruct((ep, H), jnp.float32),
          jax.ShapeDtypeStruct((ep, H), jnp.float32),
      ),
      scratch_types=[
          [pltpu.VMEM((CK,), jnp.int32)] * 4,
          [pltpu.VMEM((CK, H), jnp.float32)] * 4,
          [pltpu.SemaphoreType.DMA] * 12,
      ],
  )
  def k(xs_hbm, xr_hbm, sidx_hbm, ridx_hbm, outs_hbm, outr_hbm, ib, gb,
        sem):
    wid = lax.axis_index("s") * NC + lax.axis_index("c")
    ebase = wid * nch * CK

    def i_desc(ci, b):
      off = ebase + ci * CK
      return (pltpu.make_async_copy(sidx_hbm.at[pl.ds(off, CK)], ib[b],
                                    sem[b]),
              pltpu.make_async_copy(ridx_hbm.at[pl.ds(off, CK)], ib[2 + b],
                                    sem[2 + b]))

    def g_desc(b):
      return (pltpu.make_async_copy(xs_hbm.at[ib[b]], gb[b], sem[4 + b]),
              pltpu.make_async_copy(xr_hbm.at[ib[2 + b]], gb[2 + b],
                                    sem[6 + b]))

    def w_desc(ci, b):
      off = ebase + ci * CK
      return (pltpu.make_async_copy(gb[b], outs_hbm.at[pl.ds(off, CK)],
                                    sem[8 + b]),
              pltpu.make_async_copy(gb[2 + b], outr_hbm.at[pl.ds(off, CK)],
                                    sem[10 + b]))

    for d in i_desc(0, 0) + i_desc(1, 1):
      d.start()

    def body(g, _):
      for b in (0, 1):
        ci = 2 * g + b
        for d in i_desc(ci, b):
          d.wait()

        @pl.when(ci >= 2)
        def _():
          for d in w_desc(ci - 2, b):
            d.wait()

        for d in g_desc(b):
          d.start()
        for d in g_desc(b):
          d.wait()

        @pl.when(ci + 2 < nch)
        def _():
          for d in i_desc(ci + 2, b):
            d.start()

        for d in w_desc(ci, b):
          d.start()
      return ()

    lax.fori_loop(0, nch // 2, body, ())
    for d in w_desc(nch - 2, 0) + w_desc(nch - 1, 1):
      d.wait()

  return k(xs, xr, sidx, ridx)


@functools.partial(jax.jit, static_argnums=(2, 3))
def _sc_scatter(msg, ridx, ep, nrec):
  nh = nrec // NC
  accr = ((nh + 64) + 1023) // 1024 * 1024
  rpt = accr // NS
  nzc = rpt // 64
  nch = ep // NS // CK
  ock = 200
  noc = nh // ock
  nopt = (noc + NS - 1) // NS

  assert nch % 2 == 0

  @functools.partial(
      pl.kernel,
      mesh=_mesh(),
      out_type=jax.ShapeDtypeStruct((nrec, H), jnp.float32),
      scratch_types=[
          pltpu.VMEM((64, H), jnp.float32),
          [pltpu.VMEM((CK,), jnp.int32)] * 4,
          [pltpu.VMEM((CK, H), jnp.float32)] * 2,
          pltpu.VMEM_SHARED((accr, H), jnp.float32),
          [pltpu.SemaphoreType.DMA] * 6,
      ],
  )
  def k(msg_hbm, ridx_hbm, out_hbm, zb_v, ib, mb, acc, sem):
    c = lax.axis_index("c")
    t = lax.axis_index("s")

    def zb(i, _):
      zb_v[i // 8, pl.ds((i % 8) * 16, 16)] = jnp.zeros((16,), jnp.float32)
      return ()
    lax.fori_loop(0, 64 * 8, zb, ())

    zbase = t * rpt
    def zc(i, _):
      pltpu.sync_copy(zb_v, acc.at[pl.ds(zbase + i * 64, 64)])
      return ()
    lax.fori_loop(0, nzc, zc, ())

    lo = c * nh
    ebase = t * nch * CK
    li = [ib[2], ib[3]]

    def i_desc(ci, b):
      return pltpu.make_async_copy(
          ridx_hbm.at[pl.ds(ebase + ci * CK, CK)], ib[b], sem[b])

    def m_desc(ci, b):
      return pltpu.make_async_copy(
          msg_hbm.at[pl.ds(ebase + ci * CK, CK)], mb[b], sem[2 + b])

    def s_desc(b):
      return pltpu.make_async_copy(mb[b], acc.at[li[b]], sem[4 + b])

    i_desc(0, 0).start()
    m_desc(0, 0).start()
    plsc.subcore_barrier()

    def body(g, _):
      for b in (0, 1):
        ci = 2 * g + b
        nb = 1 - b
        i_desc(ci, b).wait()
        m_desc(ci, b).wait()

        def fix(j, _):
          v = ib[b][pl.ds(j * 16, 16)]
          lv = v - lo
          ok = (lv >= 0) & (lv < nh)
          li[b][pl.ds(j * 16, 16)] = jnp.where(ok, lv, nh)
          return ()
        lax.fori_loop(0, CK // 16, fix, ())

        @pl.when(ci >= 1)
        def _():
          s_desc(nb).wait()

        pltpu.async_copy(mb[b], acc.at[li[b]], sem[4 + b], add=True)

        @pl.when(ci + 1 < nch)
        def _():
          i_desc(ci + 1, nb).start()
          m_desc(ci + 1, nb).start()
      return ()

    lax.fori_loop(0, nch // 2, body, ())
    s_desc(1).wait()
    plsc.subcore_barrier()

    def oc(i, _):
      ch = t + i * NS
      @pl.when(ch < noc)
      def _():
        pltpu.sync_copy(acc.at[pl.ds(ch * ock, ock)],
                        out_hbm.at[pl.ds(lo + ch * ock, ock)])
      return ()
    lax.fori_loop(0, nopt, oc, ())

  return k(msg, ridx)


def _edge_body(send_b, rec_b, invt_b, w1s_b, w1r_b, w1i_b, b1_b, w2_b, b2_b,
               ewt_b, eb_b, out_b):
  f32 = jnp.float32
  h = (jnp.dot(send_b[...], w1s_b[...], preferred_element_type=f32)
       + jnp.dot(rec_b[...], w1r_b[...], preferred_element_type=f32)
       + lax.dot_general(invt_b[...], w1i_b[...],
                         (((0,), (0,)), ((), ())), preferred_element_type=f32)
       + b1_b[...])
  h = h * jax.nn.sigmoid(h)
  m = jnp.dot(h, w2_b[...], preferred_element_type=f32) + b2_b[...]
  m = m * jax.nn.sigmoid(m)
  wg = jax.nn.sigmoid(jnp.sum(m * ewt_b[...], axis=1, keepdims=True)
                      + eb_b[0])
  out_b[...] = m * wg


@functools.partial(jax.jit, static_argnums=(11,))
def _edge_mlp(send, rec, invt, w1s, w1r, w1i, b1, w2, b2, ewt, eb, ep):
  be = 1024
  grid = (ep // be,)
  wspec = pl.BlockSpec((H, H), lambda i: (0, 0))
  bspec = pl.BlockSpec((1, H), lambda i: (0, 0))
  return pl.pallas_call(
      _edge_body,
      grid=grid,
      in_specs=[
          pl.BlockSpec((be, H), lambda i: (i, 0)),
          pl.BlockSpec((be, H), lambda i: (i, 0)),
          pl.BlockSpec((8, be), lambda i: (0, i)),
          wspec, wspec,
          pl.BlockSpec((8, H), lambda i: (0, 0)),
          bspec, wspec, bspec, bspec,
          pl.BlockSpec(memory_space=pltpu.SMEM),
      ],
      out_specs=pl.BlockSpec((be, H), lambda i: (i, 0)),
      out_shape=jax.ShapeDtypeStruct((ep, H), jnp.float32),
  )(send, rec, invt, w1s, w1r, w1i, b1, w2, b2, ewt, eb)


def _node0_body(x_b, m_b, a_b, bq_b, b1_b, w2_b, b2_b, sw_b, sb_b, out_b):
  f32 = jnp.float32
  h = (jnp.dot(x_b[...], a_b[...], preferred_element_type=f32)
       + jnp.dot(m_b[...], bq_b[...], preferred_element_type=f32)
       + b1_b[...])
  h = h * jax.nn.sigmoid(h)
  out_b[...] = (jnp.dot(h, w2_b[...], preferred_element_type=f32) + b2_b[...]
                + jnp.dot(x_b[...], sw_b[...], preferred_element_type=f32)
                + sb_b[...])


def _node1_body(x_b, m1_b, m2_b, a_b, bq_b, c_b, b1_b, w2_b, b2_b, sw_b,
                sb_b, out_b):
  f32 = jnp.float32
  h = (jnp.dot(x_b[...], a_b[...], preferred_element_type=f32)
       + jnp.dot(m1_b[...], bq_b[...], preferred_element_type=f32)
       + jnp.dot(m2_b[...], c_b[...], preferred_element_type=f32)
       + b1_b[...])
  h = h * jax.nn.sigmoid(h)
  out_b[...] = (jnp.dot(h, w2_b[...], preferred_element_type=f32) + b2_b[...]
                + jnp.dot(x_b[...], sw_b[...], preferred_element_type=f32)
                + sb_b[...])


@jax.jit
def _node0(x, m, a, bq, b1, w2, b2, sw, sb):
  n = x.shape[0]
  bn = 1000
  wspec = pl.BlockSpec((H, H), lambda i: (0, 0))
  bspec = pl.BlockSpec((1, H), lambda i: (0, 0))
  nspec = pl.BlockSpec((bn, H), lambda i: (i, 0))
  return pl.pallas_call(
      _node0_body,
      grid=(n // bn,),
      in_specs=[nspec, nspec, wspec, wspec, bspec, wspec, bspec, wspec,
                bspec],
      out_specs=nspec,
      out_shape=jax.ShapeDtypeStruct((n, H), jnp.float32),
  )(x, m, a, bq, b1, w2, b2, sw, sb)


@jax.jit
def _node1(x, m1, m2, a, bq, c, b1, w2, b2, sw, sb):
  n = x.shape[0]
  bn = 1000
  wspec = pl.BlockSpec((H, H), lambda i: (0, 0))
  bspec = pl.BlockSpec((1, H), lambda i: (0, 0))
  nspec = pl.BlockSpec((bn, H), lambda i: (i, 0))
  return pl.pallas_call(
      _node1_body,
      grid=(n // bn,),
      in_specs=[nspec, nspec, nspec, wspec, wspec, wspec, bspec, wspec,
                bspec, wspec, bspec],
      out_specs=nspec,
      out_shape=jax.ShapeDtypeStruct((n, H), jnp.float32),
  )(x, m1, m2, a, bq, c, b1, w2, b2, sw, sb)


def _pad_to(x, n, val):
  e = x.shape[0]
  if e == n:
    return x
  return jnp.concatenate(
      [x, jnp.full((n - e,) + x.shape[1:], val, x.dtype)], axis=0)


def _message(xs, xr, adj, inv, w1, b1, w2, b2, ew, eb, nrec):
  e = adj.shape[1]
  ep = -(-e // (NW * CK * 2)) * (NW * CK * 2)
  ninv = inv.shape[1]
  sidx = _pad_to(adj[0].astype(jnp.int32), ep, 0)
  gidx = _pad_to(adj[1].astype(jnp.int32), ep, 0)
  ridx = _pad_to(adj[1].astype(jnp.int32), ep, BIGIDX)
  invt = jnp.zeros((8, ep), jnp.float32).at[:ninv, :e].set(inv.T)
  w1s = w1[:H]
  w1r = w1[H:2 * H]
  w1i = jnp.zeros((8, H), jnp.float32).at[:ninv].set(w1[2 * H:])
  send, rec = _sc_gather(xs, xr, sidx, gidx, ep)
  msg = _edge_mlp(send, rec, invt, w1s, w1r, w1i, b1.reshape(1, H), w2,
                  b2.reshape(1, H), ew.reshape(1, H), eb, ep)
  return _sc_scatter(msg, ridx, ep, nrec)


def kernel(x0, x1, adj_0_0, adj_0_1, adj_1_1, inv_0_0, inv_0_1, inv_1_1,
           mw1_0_0, mb1_0_0, mw2_0_0, mb2_0_0, ew_0_0, eb_0_0,
           mw1_0_1, mb1_0_1, mw2_0_1, mb2_0_1, ew_0_1, eb_0_1,
           mw1_1_1, mb1_1_1, mw2_1_1, mb2_1_1, ew_1_1, eb_1_1,
           u0w1, u0b1, u0w2, u0b2, u1w1, u1b1, u1w2, u1b2, sw, sb):
  m00 = _message(x0, x0, adj_0_0, inv_0_0, mw1_0_0, mb1_0_0, mw2_0_0,
                 mb2_0_0, ew_0_0, eb_0_0, x0.shape[0])
  m01 = _message(x0, x1, adj_0_1, inv_0_1, mw1_0_1, mb1_0_1, mw2_0_1,
                 mb2_0_1, ew_0_1, eb_0_1, x1.shape[0])
  m11 = _message(x1, x1, adj_1_1, inv_1_1, mw1_1_1, mb1_1_1, mw2_1_1,
                 mb2_1_1, ew_1_1, eb_1_1, x1.shape[0])
  out0 = _node0(x0, m00, u0w1[:H], u0w1[H:], u0b1.reshape(1, H), u0w2,
                u0b2.reshape(1, H), sw, sb.reshape(1, H))
  out1 = _node1(x1, m01, m11, u1w1[:H], u1w1[H:2 * H], u1w1[2 * H:],
                u1b1.reshape(1, H), u1w2, u1b2.reshape(1, H), sw,
                sb.reshape(1, H))
  return (out0, out1)
